# Initial kernel scaffold; baseline (speedup 1.0000x reference)
import jax
import jax.numpy as jnp
from jax.experimental import pallas as pl


def kernel(x_mark, minute_tab, hour_tab, weekday_tab, day_tab, month_tab):
    raise NotImplementedError("write your pallas kernel here")


# speedup vs baseline: 19.7554x; 19.7554x over previous
import functools

import jax
import jax.numpy as jnp
from jax.experimental import pallas as pl
from jax.experimental.pallas import tpu as pltpu

B, L, D = 4096, 200, 128
N = B * L
BLK = 2048


def _body(x_ref, tab_ref, o_ref):
    idx = x_ref[...]
    lane = jax.lax.broadcasted_iota(jnp.int32, (BLK, 128), 1)
    acc = jnp.zeros((BLK, 128), jnp.float32)
    offs = (0, 13, 45, 52, 76)
    for t in range(5):
        acc += (lane == (idx[:, t][:, None] + offs[t])).astype(jnp.float32)
    o_ref[...] = jnp.dot(acc, tab_ref[...], preferred_element_type=jnp.float32)


@jax.jit
def kernel(x_mark, minute_tab, hour_tab, weekday_tab, day_tab, month_tab):
    stacked = jnp.zeros((128, D), jnp.float32)
    stacked = jax.lax.dynamic_update_slice(stacked, month_tab, (0, 0))
    stacked = jax.lax.dynamic_update_slice(stacked, day_tab, (13, 0))
    stacked = jax.lax.dynamic_update_slice(stacked, weekday_tab, (45, 0))
    stacked = jax.lax.dynamic_update_slice(stacked, hour_tab, (52, 0))
    stacked = jax.lax.dynamic_update_slice(stacked, minute_tab, (76, 0))

    x = x_mark.reshape(N, 5).astype(jnp.int32)
    out = pl.pallas_call(
        _body,
        grid=(N // BLK,),
        in_specs=[
            pl.BlockSpec((BLK, 5), lambda i: (i, 0)),
            pl.BlockSpec((128, D), lambda i: (0, 0)),
        ],
        out_specs=pl.BlockSpec((BLK, D), lambda i: (i, 0)),
        out_shape=jax.ShapeDtypeStruct((N, D), jnp.float32),
    )(x, stacked)
    return out.reshape(B, L, D)
